# baseline (speedup 1.0000x reference)
import functools

import jax
import jax.numpy as jnp
from jax import lax
from jax.experimental import pallas as pl
from jax.experimental.pallas import tpu as pltpu
from jax.experimental.pallas import tpu_sc as plsc

B = 16384
DU = 128
DS = 16
NSMALL = 1000
IDXC = 128
ROWS_2D = B // IDXC


def _sc_info():
    info = plsc.get_sparse_core_info()
    return info.num_cores, info.num_subcores


def _make_gather():
    NC, NS = _sc_info()
    NW = NC * NS
    BPW = B // NW
    RPW = BPW // IDXC
    GPW = BPW // 16
    mesh = plsc.VectorSubcoreMesh(core_axis_name="c", subcore_axis_name="s")

    @functools.partial(
        pl.kernel,
        mesh=mesh,
        out_type=[
            jax.ShapeDtypeStruct((B, 2 * DU), jnp.float32),
            jax.ShapeDtypeStruct((3 * DS, B), jnp.float32),
        ],
        scratch_types=[
            pltpu.VMEM((8, IDXC), jnp.int32),
            pltpu.VMEM((BPW // 2, DU), jnp.float32),
            pltpu.VMEM((NSMALL * DS,), jnp.float32),
            pltpu.VMEM((NSMALL * DS,), jnp.float32),
            pltpu.VMEM((NSMALL * DS,), jnp.float32),
            pltpu.VMEM((BPW,), jnp.int32),
            pltpu.VMEM((BPW,), jnp.int32),
            pltpu.VMEM((BPW,), jnp.int32),
            pltpu.VMEM((3 * DS, BPW), jnp.float32),
            pltpu.SemaphoreType.DMA,
            pltpu.SemaphoreType.DMA,
        ],
        compiler_params=pltpu.CompilerParams(needs_layout_passes=False),
    )
    def gather(uidx, iidx, tidx, cidx, sidx, utab, itab, ttab, ctab, stab,
               ui_out, sm_out,
               idx_v, big_v, tab_vt, tab_vc, tab_vs,
               sidx_vt, sidx_vc, sidx_vs,
               smT, sem, sem2):
        wid = lax.axis_index("s") * NC + lax.axis_index("c")
        base = wid * BPW
        r0 = (wid // 2) * 8
        j0 = (wid % 2) * RPW

        stage = [
            pltpu.async_copy(ttab, tab_vt, sem2),
            pltpu.async_copy(ctab, tab_vc, sem2),
            pltpu.async_copy(stab, tab_vs, sem2),
            pltpu.async_copy(tidx.at[pl.ds(base, BPW)], sidx_vt, sem2),
            pltpu.async_copy(cidx.at[pl.ds(base, BPW)], sidx_vc, sem2),
            pltpu.async_copy(sidx.at[pl.ds(base, BPW)], sidx_vs, sem2),
        ]

        def big_gather(idx_hbm, tab_hbm, col):
            pltpu.sync_copy(idx_hbm.at[pl.ds(r0, 8)], idx_v)
            half = RPW // 2
            for h in range(2):
                cps = [
                    pltpu.async_copy(
                        tab_hbm.at[idx_v.at[j0 + h * half + j]],
                        big_v.at[pl.ds(j * IDXC, IDXC)],
                        sem,
                    )
                    for j in range(half)
                ]
                for cp in cps:
                    cp.wait()
                pltpu.sync_copy(
                    big_v,
                    ui_out.at[pl.ds(base + h * (BPW // 2), BPW // 2),
                              pl.ds(col, DU)])

        big_gather(uidx, utab, 0)
        big_gather(iidx, itab, DU)

        for cp in stage:
            cp.wait()

        def small_body(g, _):
            for r, (tab_v, sidx_v) in enumerate(((tab_vt, sidx_vt),
                                                 (tab_vc, sidx_vc),
                                                 (tab_vs, sidx_vs))):
                idx16 = sidx_v[pl.ds(g * 16, 16)]
                flat = idx16 * DS
                for k in range(DS):
                    vals = plsc.load_gather(tab_v, [flat + k])
                    smT[r * DS + k, pl.ds(g * 16, 16)] = vals
            return 0

        lax.fori_loop(0, GPW, small_body, 0)

        pltpu.sync_copy(smT, sm_out.at[:, pl.ds(base, BPW)])

    return gather


_gather = None


def _get_gather():
    global _gather
    if _gather is None:
        _gather = _make_gather()
    return _gather


BBA = 1024
BBB = 1024


def _mlp_a_body(text, image, w1a, b1, out_ref):
    f32 = jnp.float32
    acc = jnp.dot(image[...], w1a[pl.ds(384, 2048), :],
                  preferred_element_type=f32)
    acc = acc + jnp.dot(text[...], w1a[pl.ds(0, 384), :],
                        preferred_element_type=f32)
    out_ref[...] = acc + b1[...]


def _mlp_a(text, image, w1a, b1):
    def whole(a):
        shp = a.shape
        return pl.BlockSpec(shp, lambda g: tuple(0 for _ in shp))

    in_specs = [
        pl.BlockSpec((BBA, 384), lambda g: (g, 0)),
        pl.BlockSpec((BBA, 2048), lambda g: (g, 0)),
        whole(w1a), whole(b1),
    ]
    return pl.pallas_call(
        _mlp_a_body,
        grid=(B // BBA,),
        in_specs=in_specs,
        out_specs=pl.BlockSpec((BBA, 256), lambda g: (g, 0)),
        out_shape=jax.ShapeDtypeStruct((B, 256), jnp.float32),
        compiler_params=pltpu.CompilerParams(
            dimension_semantics=("arbitrary",),
        ),
    )(text, image, w1a, b1)


_SMALL_DN = (((0,), (0,)), ((), ()))


def _mlp_b_body(part, ui, sm, w1, w2, b2, w3, b3, out_ref):
    f32 = jnp.float32
    acc = part[...]
    acc = acc + jnp.dot(ui[...], w1[pl.ds(0, 256), :],
                        preferred_element_type=f32)
    acc = acc + lax.dot_general(sm[...], w1[pl.ds(256, 48), :], _SMALL_DN,
                                preferred_element_type=f32)
    h1 = jnp.maximum(acc, 0.0)
    h2 = jnp.maximum(
        jnp.dot(h1, w2[...], preferred_element_type=f32) + b2[...], 0.0)
    out = jnp.sum(h2 * w3[...], axis=1) + b3[0, 0]
    out_ref[...] = out


def _mlp_b(part, ui, sm, w1, w2, b2, w3, b3):
    def whole(a):
        shp = a.shape
        return pl.BlockSpec(shp, lambda g: tuple(0 for _ in shp))

    w1b = pl.BlockSpec((304, 256), lambda g: (0, 0))
    in_specs = [
        pl.BlockSpec((BBB, 256), lambda g: (g, 0)),
        pl.BlockSpec((BBB, 2 * DU), lambda g: (g, 0)),
        pl.BlockSpec((3 * DS, BBB), lambda g: (0, g)),
        w1b, whole(w2), whole(b2), whole(w3), whole(b3),
    ]
    return pl.pallas_call(
        _mlp_b_body,
        grid=(B // BBB,),
        in_specs=in_specs,
        out_specs=pl.BlockSpec((BBB,), lambda g: (g,)),
        out_shape=jax.ShapeDtypeStruct((B,), jnp.float32),
        compiler_params=pltpu.CompilerParams(
            dimension_semantics=("arbitrary",),
        ),
    )(part, ui, sm, w1, w2, b2, w3, b3)


def kernel(user, item, type_idx, color_idx, section_idx, text_vec, image_vec,
           user_table, item_table, type_table, color_table, section_table,
           W1, b1, W2, b2, W3, b3):
    ui = user.astype(jnp.int32).reshape(ROWS_2D, IDXC)
    ii = item.astype(jnp.int32).reshape(ROWS_2D, IDXC)
    ti = type_idx.astype(jnp.int32)
    ci = color_idx.astype(jnp.int32)
    si = section_idx.astype(jnp.int32)

    ui_rows, sm_rows = _get_gather()(
        ui, ii, ti, ci, si, user_table, item_table,
        type_table.reshape(NSMALL * DS),
        color_table.reshape(NSMALL * DS),
        section_table.reshape(NSMALL * DS))

    bf16 = jnp.bfloat16
    part = _mlp_a(text_vec.astype(bf16), image_vec.astype(bf16),
                  W1[304:].astype(bf16), b1.reshape(1, 256))
    return _mlp_b(
        part, ui_rows, sm_rows, W1,
        W2, b2.reshape(1, 128), W3.reshape(1, 128), b3.reshape(1, 1))


# speedup vs baseline: 1.4496x; 1.4496x over previous
import functools

import jax
import jax.numpy as jnp
from jax import lax
from jax.experimental import pallas as pl
from jax.experimental.pallas import tpu as pltpu
from jax.experimental.pallas import tpu_sc as plsc

B = 16384
DU = 128
DS = 16
NSMALL = 1000
IDXC = 128
ROWS_2D = B // IDXC


def _sc_info():
    info = plsc.get_sparse_core_info()
    return info.num_cores, info.num_subcores


def _make_gather():
    NC, NS = _sc_info()
    NW = NC * NS
    BPW = B // NW
    RPW = BPW // IDXC
    GPW = BPW // 16
    mesh = plsc.VectorSubcoreMesh(core_axis_name="c", subcore_axis_name="s")

    @functools.partial(
        pl.kernel,
        mesh=mesh,
        out_type=[
            jax.ShapeDtypeStruct((B, 2 * DU), jnp.float32),
            jax.ShapeDtypeStruct((3 * DS, B), jnp.float32),
        ],
        scratch_types=[
            pltpu.VMEM((8, IDXC), jnp.int32),
            pltpu.VMEM((BPW // 2, DU), jnp.float32),
            pltpu.VMEM((NSMALL * DS,), jnp.float32),
            pltpu.VMEM((NSMALL * DS,), jnp.float32),
            pltpu.VMEM((NSMALL * DS,), jnp.float32),
            pltpu.VMEM((BPW,), jnp.int32),
            pltpu.VMEM((BPW,), jnp.int32),
            pltpu.VMEM((BPW,), jnp.int32),
            pltpu.VMEM((3 * DS, BPW), jnp.float32),
            pltpu.SemaphoreType.DMA,
            pltpu.SemaphoreType.DMA,
        ],
        compiler_params=pltpu.CompilerParams(needs_layout_passes=False),
    )
    def gather(uidx, iidx, tidx, cidx, sidx, utab, itab, ttab, ctab, stab,
               ui_out, sm_out,
               idx_v, big_v, tab_vt, tab_vc, tab_vs,
               sidx_vt, sidx_vc, sidx_vs,
               smT, sem, sem2):
        wid = lax.axis_index("s") * NC + lax.axis_index("c")
        base = wid * BPW
        r0 = (wid // 2) * 8
        j0 = (wid % 2) * RPW

        stage = [
            pltpu.async_copy(ttab, tab_vt, sem2),
            pltpu.async_copy(ctab, tab_vc, sem2),
            pltpu.async_copy(stab, tab_vs, sem2),
            pltpu.async_copy(tidx.at[pl.ds(base, BPW)], sidx_vt, sem2),
            pltpu.async_copy(cidx.at[pl.ds(base, BPW)], sidx_vc, sem2),
            pltpu.async_copy(sidx.at[pl.ds(base, BPW)], sidx_vs, sem2),
        ]

        def big_gather(idx_hbm, tab_hbm, col):
            pltpu.sync_copy(idx_hbm.at[pl.ds(r0, 8)], idx_v)
            half = RPW // 2
            for h in range(2):
                cps = [
                    pltpu.async_copy(
                        tab_hbm.at[idx_v.at[j0 + h * half + j]],
                        big_v.at[pl.ds(j * IDXC, IDXC)],
                        sem,
                    )
                    for j in range(half)
                ]
                for cp in cps:
                    cp.wait()
                pltpu.sync_copy(
                    big_v,
                    ui_out.at[pl.ds(base + h * (BPW // 2), BPW // 2),
                              pl.ds(col, DU)])

        big_gather(uidx, utab, 0)
        big_gather(iidx, itab, DU)

        for cp in stage:
            cp.wait()

        def small_body(g, _):
            for r, (tab_v, sidx_v) in enumerate(((tab_vt, sidx_vt),
                                                 (tab_vc, sidx_vc),
                                                 (tab_vs, sidx_vs))):
                idx16 = sidx_v[pl.ds(g * 16, 16)]
                flat = idx16 * DS
                for k in range(DS):
                    vals = plsc.load_gather(tab_v, [flat + k])
                    smT[r * DS + k, pl.ds(g * 16, 16)] = vals
            return 0

        lax.fori_loop(0, GPW, small_body, 0)

        pltpu.sync_copy(smT, sm_out.at[:, pl.ds(base, BPW)])

    return gather


_gather = None


def _get_gather():
    global _gather
    if _gather is None:
        _gather = _make_gather()
    return _gather


BBA = 1024
BBB = 1024


def _mlp_a_body(text, image, w1a, b1, out_ref):
    f32 = jnp.float32
    bf16 = jnp.bfloat16
    acc = jnp.dot(image[...].astype(bf16), w1a[pl.ds(384, 2048), :],
                  preferred_element_type=f32)
    acc = acc + jnp.dot(text[...].astype(bf16), w1a[pl.ds(0, 384), :],
                        preferred_element_type=f32)
    out_ref[...] = acc + b1[...]


def _mlp_a(text, image, w1a, b1):
    def whole(a):
        shp = a.shape
        return pl.BlockSpec(shp, lambda g: tuple(0 for _ in shp))

    in_specs = [
        pl.BlockSpec((BBA, 384), lambda g: (g, 0)),
        pl.BlockSpec((BBA, 2048), lambda g: (g, 0)),
        whole(w1a), whole(b1),
    ]
    return pl.pallas_call(
        _mlp_a_body,
        grid=(B // BBA,),
        in_specs=in_specs,
        out_specs=pl.BlockSpec((BBA, 256), lambda g: (g, 0)),
        out_shape=jax.ShapeDtypeStruct((B, 256), jnp.float32),
        compiler_params=pltpu.CompilerParams(
            dimension_semantics=("arbitrary",),
        ),
    )(text, image, w1a, b1)


_SMALL_DN = (((0,), (0,)), ((), ()))


def _mlp_b_body(part, ui, sm, w1, w2, b2, w3, b3, out_ref):
    f32 = jnp.float32
    acc = part[...]
    acc = acc + jnp.dot(ui[...], w1[pl.ds(0, 256), :],
                        preferred_element_type=f32)
    acc = acc + lax.dot_general(sm[...], w1[pl.ds(256, 48), :], _SMALL_DN,
                                preferred_element_type=f32)
    h1 = jnp.maximum(acc, 0.0)
    h2 = jnp.maximum(
        jnp.dot(h1, w2[...], preferred_element_type=f32) + b2[...], 0.0)
    out = jnp.sum(h2 * w3[...], axis=1) + b3[0, 0]
    out_ref[...] = out


def _mlp_b(part, ui, sm, w1, w2, b2, w3, b3):
    def whole(a):
        shp = a.shape
        return pl.BlockSpec(shp, lambda g: tuple(0 for _ in shp))

    w1b = pl.BlockSpec((304, 256), lambda g: (0, 0))
    in_specs = [
        pl.BlockSpec((BBB, 256), lambda g: (g, 0)),
        pl.BlockSpec((BBB, 2 * DU), lambda g: (g, 0)),
        pl.BlockSpec((3 * DS, BBB), lambda g: (0, g)),
        w1b, whole(w2), whole(b2), whole(w3), whole(b3),
    ]
    return pl.pallas_call(
        _mlp_b_body,
        grid=(B // BBB,),
        in_specs=in_specs,
        out_specs=pl.BlockSpec((BBB,), lambda g: (g,)),
        out_shape=jax.ShapeDtypeStruct((B,), jnp.float32),
        compiler_params=pltpu.CompilerParams(
            dimension_semantics=("arbitrary",),
        ),
    )(part, ui, sm, w1, w2, b2, w3, b3)


def kernel(user, item, type_idx, color_idx, section_idx, text_vec, image_vec,
           user_table, item_table, type_table, color_table, section_table,
           W1, b1, W2, b2, W3, b3):
    ui = user.astype(jnp.int32).reshape(ROWS_2D, IDXC)
    ii = item.astype(jnp.int32).reshape(ROWS_2D, IDXC)
    ti = type_idx.astype(jnp.int32)
    ci = color_idx.astype(jnp.int32)
    si = section_idx.astype(jnp.int32)

    ui_rows, sm_rows = _get_gather()(
        ui, ii, ti, ci, si, user_table, item_table,
        type_table.reshape(NSMALL * DS),
        color_table.reshape(NSMALL * DS),
        section_table.reshape(NSMALL * DS))

    part = _mlp_a(text_vec, image_vec,
                  W1[304:].astype(jnp.bfloat16), b1.reshape(1, 256))
    return _mlp_b(
        part, ui_rows, sm_rows, W1,
        W2, b2.reshape(1, 128), W3.reshape(1, 128), b3.reshape(1, 1))


# speedup vs baseline: 1.4775x; 1.0192x over previous
import functools

import jax
import jax.numpy as jnp
from jax import lax
from jax.experimental import pallas as pl
from jax.experimental.pallas import tpu as pltpu
from jax.experimental.pallas import tpu_sc as plsc

B = 16384
DU = 128
DS = 16
NSMALL = 1000
IDXC = 128
ROWS_2D = B // IDXC


def _sc_info():
    info = plsc.get_sparse_core_info()
    return info.num_cores, info.num_subcores


def _make_gather():
    NC, NS = _sc_info()
    NW = NC * NS
    BPW = B // NW
    RPW = BPW // IDXC
    GPW = BPW // 16
    mesh = plsc.VectorSubcoreMesh(core_axis_name="c", subcore_axis_name="s")

    @functools.partial(
        pl.kernel,
        mesh=mesh,
        out_type=[
            jax.ShapeDtypeStruct((B, 2 * DU), jnp.float32),
            jax.ShapeDtypeStruct((3 * DS, B), jnp.float32),
        ],
        scratch_types=[
            pltpu.VMEM((8, IDXC), jnp.int32),
            pltpu.VMEM((BPW // 2, DU), jnp.float32),
            pltpu.VMEM((NSMALL * DS,), jnp.float32),
            pltpu.VMEM((NSMALL * DS,), jnp.float32),
            pltpu.VMEM((NSMALL * DS,), jnp.float32),
            pltpu.VMEM((BPW,), jnp.int32),
            pltpu.VMEM((BPW,), jnp.int32),
            pltpu.VMEM((BPW,), jnp.int32),
            pltpu.VMEM((3 * DS, BPW), jnp.float32),
            pltpu.SemaphoreType.DMA,
            pltpu.SemaphoreType.DMA,
        ],
        compiler_params=pltpu.CompilerParams(needs_layout_passes=False),
    )
    def gather(uidx, iidx, tidx, cidx, sidx, utab, itab, ttab, ctab, stab,
               ui_out, sm_out,
               idx_v, big_v, tab_vt, tab_vc, tab_vs,
               sidx_vt, sidx_vc, sidx_vs,
               smT, sem, sem2):
        wid = lax.axis_index("s") * NC + lax.axis_index("c")
        base = wid * BPW
        r0 = (wid // 2) * 8
        j0 = (wid % 2) * RPW

        stage = [
            pltpu.async_copy(ttab, tab_vt, sem2),
            pltpu.async_copy(ctab, tab_vc, sem2),
            pltpu.async_copy(stab, tab_vs, sem2),
            pltpu.async_copy(tidx.at[pl.ds(base, BPW)], sidx_vt, sem2),
            pltpu.async_copy(cidx.at[pl.ds(base, BPW)], sidx_vc, sem2),
            pltpu.async_copy(sidx.at[pl.ds(base, BPW)], sidx_vs, sem2),
        ]

        def big_gather(idx_hbm, tab_hbm, col):
            pltpu.sync_copy(idx_hbm.at[pl.ds(r0, 8)], idx_v)
            half = RPW // 2
            for h in range(2):
                cps = [
                    pltpu.async_copy(
                        tab_hbm.at[idx_v.at[j0 + h * half + j]],
                        big_v.at[pl.ds(j * IDXC, IDXC)],
                        sem,
                    )
                    for j in range(half)
                ]
                for cp in cps:
                    cp.wait()
                pltpu.sync_copy(
                    big_v,
                    ui_out.at[pl.ds(base + h * (BPW // 2), BPW // 2),
                              pl.ds(col, DU)])

        big_gather(uidx, utab, 0)
        big_gather(iidx, itab, DU)

        for cp in stage:
            cp.wait()

        def small_body(g, _):
            for r, (tab_v, sidx_v) in enumerate(((tab_vt, sidx_vt),
                                                 (tab_vc, sidx_vc),
                                                 (tab_vs, sidx_vs))):
                idx16 = sidx_v[pl.ds(g * 16, 16)]
                flat = idx16 * DS
                for k in range(DS):
                    vals = plsc.load_gather(tab_v, [flat + k])
                    smT[r * DS + k, pl.ds(g * 16, 16)] = vals
            return 0

        lax.fori_loop(0, GPW, small_body, 0)

        pltpu.sync_copy(smT, sm_out.at[:, pl.ds(base, BPW)])

    return gather


_gather = None


def _get_gather():
    global _gather
    if _gather is None:
        _gather = _make_gather()
    return _gather


BBA = 1024
BBB = 1024


def _mlp_a_body(text, image, w1a, b1, out_ref):
    f32 = jnp.float32
    bf16 = jnp.bfloat16
    acc = jnp.dot(image[...].astype(bf16), w1a[pl.ds(384, 2048), :],
                  preferred_element_type=f32)
    acc = acc + jnp.dot(text[...].astype(bf16), w1a[pl.ds(0, 384), :],
                        preferred_element_type=f32)
    out_ref[...] = (acc + b1[...]).astype(bf16)


def _mlp_a(text, image, w1a, b1):
    def whole(a):
        shp = a.shape
        return pl.BlockSpec(shp, lambda g: tuple(0 for _ in shp))

    in_specs = [
        pl.BlockSpec((BBA, 384), lambda g: (g, 0)),
        pl.BlockSpec((BBA, 2048), lambda g: (g, 0)),
        whole(w1a), whole(b1),
    ]
    return pl.pallas_call(
        _mlp_a_body,
        grid=(B // BBA,),
        in_specs=in_specs,
        out_specs=pl.BlockSpec((BBA, 256), lambda g: (g, 0)),
        out_shape=jax.ShapeDtypeStruct((B, 256), jnp.bfloat16),
        compiler_params=pltpu.CompilerParams(
            dimension_semantics=("arbitrary",),
        ),
    )(text, image, w1a, b1)


_SMALL_DN = (((0,), (0,)), ((), ()))


def _mlp_b_body(part, ui, sm, w1, w2, b2, w3, b3, out_ref):
    f32 = jnp.float32
    bf16 = jnp.bfloat16
    acc = part[...].astype(f32)
    acc = acc + jnp.dot(ui[...].astype(bf16), w1[pl.ds(0, 256), :],
                        preferred_element_type=f32)
    acc = acc + lax.dot_general(sm[...].astype(bf16), w1[pl.ds(256, 48), :],
                                _SMALL_DN, preferred_element_type=f32)
    h1 = jnp.maximum(acc, 0.0).astype(bf16)
    h2 = jnp.maximum(
        jnp.dot(h1, w2[...], preferred_element_type=f32) + b2[...], 0.0)
    out = jnp.sum(h2 * w3[...], axis=1) + b3[0, 0]
    out_ref[...] = out


def _mlp_b(part, ui, sm, w1, w2, b2, w3, b3):
    def whole(a):
        shp = a.shape
        return pl.BlockSpec(shp, lambda g: tuple(0 for _ in shp))

    w1b = pl.BlockSpec((304, 256), lambda g: (0, 0))
    in_specs = [
        pl.BlockSpec((BBB, 256), lambda g: (g, 0)),
        pl.BlockSpec((BBB, 2 * DU), lambda g: (g, 0)),
        pl.BlockSpec((3 * DS, BBB), lambda g: (0, g)),
        w1b, whole(w2), whole(b2), whole(w3), whole(b3),
    ]
    return pl.pallas_call(
        _mlp_b_body,
        grid=(B // BBB,),
        in_specs=in_specs,
        out_specs=pl.BlockSpec((BBB,), lambda g: (g,)),
        out_shape=jax.ShapeDtypeStruct((B,), jnp.float32),
        compiler_params=pltpu.CompilerParams(
            dimension_semantics=("arbitrary",),
        ),
    )(part, ui, sm, w1, w2, b2, w3, b3)


def kernel(user, item, type_idx, color_idx, section_idx, text_vec, image_vec,
           user_table, item_table, type_table, color_table, section_table,
           W1, b1, W2, b2, W3, b3):
    ui = user.astype(jnp.int32).reshape(ROWS_2D, IDXC)
    ii = item.astype(jnp.int32).reshape(ROWS_2D, IDXC)
    ti = type_idx.astype(jnp.int32)
    ci = color_idx.astype(jnp.int32)
    si = section_idx.astype(jnp.int32)

    ui_rows, sm_rows = _get_gather()(
        ui, ii, ti, ci, si, user_table, item_table,
        type_table.reshape(NSMALL * DS),
        color_table.reshape(NSMALL * DS),
        section_table.reshape(NSMALL * DS))

    bf16 = jnp.bfloat16
    part = _mlp_a(text_vec, image_vec,
                  W1[304:].astype(bf16), b1.reshape(1, 256))
    return _mlp_b(
        part, ui_rows, sm_rows, W1[:304].astype(bf16),
        W2.astype(bf16), b2.reshape(1, 128), W3.reshape(1, 128),
        b3.reshape(1, 1))
